# baseline (speedup 1.0000x reference)
import functools

import jax
import jax.numpy as jnp
from jax import lax
from jax.experimental import pallas as pl
from jax.experimental.pallas import tpu as pltpu
from jax.experimental.pallas import tpu_sc as plsc

TOP_K = 8
BLOCK_T = 1024
N_EXP = 64
CH = 128


def _matmul_body(x_ref, wt_ref, lg_ref):
    lg_ref[...] = jax.lax.dot_general(
        x_ref[...], wt_ref[...],
        dimension_numbers=(((1,), (0,)), ((), ())),
        preferred_element_type=jnp.float32,
    )


def _keyify(v16, inv_iota16):
    bits = lax.bitcast_convert_type(v16, jnp.int32)
    skey = jnp.where(bits >= 0, bits, bits ^ jnp.int32(0x7FFFFFFF))
    return (skey & jnp.int32(~63)) | inv_iota16


def _sc_topk(lg_flat, tokens):
    NC, NS = 2, 16
    NW = NC * NS
    rows_w = tokens // NW
    mesh = plsc.VectorSubcoreMesh(core_axis_name="c", subcore_axis_name="s")

    @functools.partial(
        pl.kernel, mesh=mesh,
        compiler_params=pltpu.CompilerParams(needs_layout_passes=False),
        out_type=[
            jax.ShapeDtypeStruct((tokens * TOP_K,), jnp.float32),
            jax.ShapeDtypeStruct((tokens * TOP_K,), jnp.int32),
        ],
        scratch_types=[
            pltpu.VMEM((CH * N_EXP,), jnp.float32),
            pltpu.VMEM((64,), jnp.int32),
            pltpu.VMEM((CH * TOP_K,), jnp.float32),
            pltpu.VMEM((CH * TOP_K,), jnp.int32),
        ],
    )
    def k(lg_hbm, vals_hbm, idxs_hbm, buf, mrg, vout, iout):
        wid = lax.axis_index("s") * NC + lax.axis_index("c")
        base = wid * rows_w
        lane = lax.iota(jnp.int32, 16)
        invs = [jnp.int32(N_EXP - 1) - (lane + 16 * q) for q in range(4)]

        def chunk_body(c, carry):
            row0 = base + c * CH
            pltpu.sync_copy(lg_hbm.at[pl.ds(row0 * N_EXP, CH * N_EXP)], buf)

            def top8_keys(off):
                ks = [_keyify(buf[pl.ds(off + 16 * q, 16)], invs[q])
                      for q in range(4)]
                s0, _ = plsc.sort_key_val(ks[0], ks[0])
                s1, _ = plsc.sort_key_val(ks[1], ks[1], descending=True)
                mrg[pl.ds(0, 16)] = s0
                mrg[pl.ds(16, 16)] = s1
                m01, _ = plsc.sort_key_val(mrg[pl.ds(8, 16)],
                                           mrg[pl.ds(8, 16)])
                s2, _ = plsc.sort_key_val(ks[2], ks[2])
                s3, _ = plsc.sort_key_val(ks[3], ks[3], descending=True)
                mrg[pl.ds(0, 16)] = s2
                mrg[pl.ds(16, 16)] = s3
                m23, _ = plsc.sort_key_val(mrg[pl.ds(8, 16)], mrg[pl.ds(8, 16)],
                                           descending=True)
                mrg[pl.ds(0, 16)] = m01
                mrg[pl.ds(16, 16)] = m23
                fin, _ = plsc.sort_key_val(mrg[pl.ds(8, 16)], mrg[pl.ds(8, 16)],
                                           descending=True)
                return fin

            def pair_body(p, carry2):
                f0 = top8_keys(p * 2 * N_EXP)
                mrg[pl.ds(32, 16)] = f0
                f1 = top8_keys(p * 2 * N_EXP + N_EXP)
                mrg[pl.ds(40, 16)] = f1
                wk = mrg[pl.ds(32, 16)]
                st = wk & jnp.int32(~63)
                vb = jnp.where(st >= 0, st, st ^ jnp.int32(0x7FFFFFFF))
                vout[pl.ds(p * 16, 16)] = lax.bitcast_convert_type(
                    vb, jnp.float32)
                iout[pl.ds(p * 16, 16)] = (
                    jnp.int32(N_EXP - 1) - (wk & jnp.int32(63)))
                return carry2

            lax.fori_loop(0, CH // 2, pair_body, 0)
            pltpu.sync_copy(vout, vals_hbm.at[pl.ds(row0 * TOP_K, CH * TOP_K)])
            pltpu.sync_copy(iout, idxs_hbm.at[pl.ds(row0 * TOP_K, CH * TOP_K)])
            return carry

        lax.fori_loop(0, rows_w // CH, chunk_body, 0)

    return k(lg_flat)


@jax.jit
def kernel(x, w_g):
    tokens, d = x.shape
    n_exp = w_g.shape[0]
    wt = w_g.T
    logits = pl.pallas_call(
        _matmul_body,
        grid=(tokens // BLOCK_T,),
        in_specs=[
            pl.BlockSpec((BLOCK_T, d), lambda i: (i, 0)),
            pl.BlockSpec((d, n_exp), lambda i: (0, 0)),
        ],
        out_specs=pl.BlockSpec((BLOCK_T, n_exp), lambda i: (i, 0)),
        out_shape=jax.ShapeDtypeStruct((tokens, n_exp), jnp.float32),
        compiler_params=pltpu.CompilerParams(
            dimension_semantics=("arbitrary",),
        ),
    )(x, wt)
    vals_flat, idxs_flat = _sc_topk(logits.reshape(-1), tokens)
    return (vals_flat.reshape(tokens, TOP_K), idxs_flat.reshape(tokens, TOP_K))


# speedup vs baseline: 1.4723x; 1.4723x over previous
import jax
import jax.numpy as jnp
from jax.experimental import pallas as pl
from jax.experimental.pallas import tpu as pltpu

TOP_K = 8
BLOCK_T = 1024

_MASKED = -2**31


def _router_body(x_ref, wt_ref, vals_ref, idxs_ref):
    logits = jax.lax.dot_general(
        x_ref[...], wt_ref[...],
        dimension_numbers=(((1,), (0,)), ((), ())),
        preferred_element_type=jnp.float32,
    )
    n_exp = logits.shape[1]
    bits = jax.lax.bitcast_convert_type(logits, jnp.int32)
    skey = jnp.where(bits >= 0, bits, bits ^ jnp.int32(0x7FFFFFFF))
    iota = jax.lax.broadcasted_iota(jnp.int32, logits.shape, 1)
    key = (skey & jnp.int32(~63)) | (jnp.int32(n_exp - 1) - iota)
    wins = []
    for _ in range(TOP_K):
        w = jnp.max(key, axis=1, keepdims=True)
        wins.append(w)
        key = jnp.where(key == w, jnp.int32(_MASKED), key)
    wk = jnp.concatenate(wins, axis=1)
    idxs_ref[...] = jnp.int32(n_exp - 1) - (wk & jnp.int32(63))
    st = wk & jnp.int32(~63)
    vb = jnp.where(st >= 0, st, st ^ jnp.int32(0x7FFFFFFF))
    vals_ref[...] = jax.lax.bitcast_convert_type(vb, jnp.float32)


@jax.jit
def kernel(x, w_g):
    tokens, d = x.shape
    n_exp = w_g.shape[0]
    wt = w_g.T
    grid = (tokens // BLOCK_T,)
    vals, idxs = pl.pallas_call(
        _router_body,
        grid=grid,
        in_specs=[
            pl.BlockSpec((BLOCK_T, d), lambda i: (i, 0)),
            pl.BlockSpec((d, n_exp), lambda i: (0, 0)),
        ],
        out_specs=[
            pl.BlockSpec((BLOCK_T, TOP_K), lambda i: (i, 0)),
            pl.BlockSpec((BLOCK_T, TOP_K), lambda i: (i, 0)),
        ],
        out_shape=[
            jax.ShapeDtypeStruct((tokens, TOP_K), jnp.float32),
            jax.ShapeDtypeStruct((tokens, TOP_K), jnp.int32),
        ],
        compiler_params=pltpu.CompilerParams(
            dimension_semantics=("arbitrary",),
        ),
    )(x, wt)
    return (vals, idxs)
